# Initial kernel scaffold; baseline (speedup 1.0000x reference)
import jax
import jax.numpy as jnp
from jax.experimental import pallas as pl


def kernel(x, edge_index, params):
    raise NotImplementedError("write your pallas kernel here")


# speedup vs baseline: 5.1083x; 5.1083x over previous
import functools

import jax
import jax.numpy as jnp
from jax import lax
from jax.experimental import pallas as pl
from jax.experimental.pallas import tpu as pltpu
from jax.experimental.pallas import tpu_sc as plsc

N = 100000
E = 1600000
LN = 16
TRASH = N
ACC_ROWS = 100096
OUTR = 100096
EG = 128
GRP = 8
E_ROWS = 12500
E_ROWS_PAD = 12800
BN_ = 2000
NB = N // BN_
f32 = jnp.float32
i32 = jnp.int32


def _sc_body(gather, c4, *refs):
    if gather:
        g_hbm, src_hbm, didx_hbm, out_hbm, sidx, didx, rows, acc, gsem, ssem = refs
    else:
        didx_hbm, out_hbm, sidx, didx, rows, acc, gsem, ssem = refs
    cid = lax.axis_index("core")
    sid = lax.axis_index("sub")

    def fill_rows(val):
        def fb(i, _):
            rows[i, :] = val
            return 0
        lax.fori_loop(0, GRP * EG, fb, 0)

    def accumulate(g_tile0, nb, chunk_off):
        def batch(bi, _):
            gb = g_tile0 + bi * GRP
            pltpu.sync_copy(didx_hbm.at[pl.ds(gb, GRP)], didx)
            if gather:
                pltpu.sync_copy(src_hbm.at[pl.ds(gb, GRP)], sidx)
                if c4:
                    for r in range(GRP):
                        for l in range(EG // 16):
                            s = pl.ds(l * 16, 16)
                            sidx[r, s] = sidx[r, s] + chunk_off
                des = [pltpu.async_copy(g_hbm.at[sidx.at[j]],
                                        rows.at[pl.ds(j * EG, EG)], gsem)
                       for j in range(GRP)]
                for d in des:
                    d.wait()
            des = [pltpu.async_copy(rows.at[pl.ds(j * EG, EG)],
                                    acc.at[didx.at[j]], ssem, add=True)
                   for j in range(GRP)]
            for d in des:
                d.wait()
            return 0
        lax.fori_loop(0, nb, batch, 0)

    def do_chunk(g_tile0, nb, chunk_off, out_base):
        fill_rows(jnp.zeros((16,), f32))
        zb = sid * (ACC_ROWS // 16)
        for j in range(6):
            pltpu.sync_copy(rows.at[pl.ds(0, 1024)],
                            acc.at[pl.ds(zb + j * 1024, 1024)])
        pltpu.sync_copy(rows.at[pl.ds(0, ACC_ROWS // 16 - 6 * 1024)],
                        acc.at[pl.ds(zb + 6 * 1024, ACC_ROWS // 16 - 6 * 1024)])
        plsc.subcore_barrier()
        if not gather:
            fill_rows(jnp.ones((16,), f32))
        accumulate(g_tile0, nb, chunk_off)
        plsc.subcore_barrier()
        rb = sid * (OUTR // 16)
        for j in range(6):
            pltpu.sync_copy(acc.at[pl.ds(rb + j * 1024, 1024)],
                            out_hbm.at[pl.ds(out_base + rb + j * 1024, 1024)])
        pltpu.sync_copy(acc.at[pl.ds(rb + 6 * 1024, OUTR // 16 - 6 * 1024)],
                        out_hbm.at[pl.ds(out_base + rb + 6 * 1024,
                                         OUTR // 16 - 6 * 1024)])
        plsc.subcore_barrier()

    if c4:
        g_tile0 = sid * (E_ROWS_PAD // 16)
        for cc in range(2):
            chunk = 2 * cid + cc
            do_chunk(g_tile0, (E_ROWS_PAD // 16) // GRP, chunk * N, chunk * OUTR)
    else:
        g_tile0 = cid * (E_ROWS_PAD // 2) + sid * (E_ROWS_PAD // 32)
        do_chunk(g_tile0, (E_ROWS_PAD // 32) // GRP, 0, cid * OUTR)


def _make_sc(gather, c4):
    out_rows = 4 * OUTR if c4 else 2 * OUTR
    mesh = plsc.VectorSubcoreMesh(core_axis_name="core", subcore_axis_name="sub")
    scratch = [
        pltpu.VMEM((GRP, EG), i32),
        pltpu.VMEM((GRP, EG), i32),
        pltpu.VMEM((GRP * EG, LN), f32),
        pltpu.VMEM_SHARED((ACC_ROWS, LN), f32),
        pltpu.SemaphoreType.DMA,
        pltpu.SemaphoreType.DMA,
    ]
    body = functools.partial(_sc_body, gather, c4)
    return pl.kernel(
        body,
        out_type=jax.ShapeDtypeStruct((out_rows, LN), f32),
        mesh=mesh,
        scratch_types=scratch,
        compiler_params=pltpu.CompilerParams(use_tc_tiling_on_sc=False),
        name=("sc_prop4" if c4 else ("sc_prop1" if gather else "sc_hist")),
    )


_sc_hist = _make_sc(gather=False, c4=False)
_sc_prop1 = _make_sc(gather=True, c4=False)
_sc_prop4 = _make_sc(gather=True, c4=True)


def _prep(src2d, dst2d):
    def body(s_ref, d_ref, dm_ref, sm_ref):
        s = s_ref[...]
        d = d_ref[...]
        m = s != d
        dm_ref[...] = jnp.where(m, d, TRASH)
        sm_ref[...] = jnp.where(m, s, TRASH)
    blk = pl.BlockSpec((E_ROWS, 128), lambda: (0, 0))
    return pl.pallas_call(
        body,
        in_specs=[blk, blk],
        out_specs=[blk, blk],
        out_shape=[jax.ShapeDtypeStruct((E_ROWS, 128), i32)] * 2,
    )(src2d, dst2d)


def _dinv(degp):
    def body(p_ref, o_ref):
        deg = p_ref[0] + p_ref[1]
        o_ref[...] = jnp.where(deg > 0.0, lax.rsqrt(jnp.maximum(deg, 1.0)), 0.0)
    return pl.pallas_call(
        body,
        grid=(NB,),
        in_specs=[pl.BlockSpec((2, BN_, LN), lambda i: (0, i, 0))],
        out_specs=pl.BlockSpec((BN_, LN), lambda i: (i, 0)),
        out_shape=jax.ShapeDtypeStruct((N, LN), f32),
    )(degp)


def _scale(a, dinv):
    def body(a_ref, v_ref, o_ref):
        o_ref[...] = a_ref[...] * v_ref[...]
    blk = pl.BlockSpec((BN_, LN), lambda i: (i, 0))
    return pl.pallas_call(
        body,
        grid=(NB,),
        in_specs=[blk, blk],
        out_specs=blk,
        out_shape=jax.ShapeDtypeStruct((N, LN), f32),
    )(a, dinv)


def _combine1(P, dinv, prev, alpha):
    has_prev = prev is not None

    def body(*refs):
        if has_prev:
            p_ref, v_ref, t_ref, tx_ref, g_ref = refs
        else:
            p_ref, v_ref, tx_ref, g_ref = refs
        v = v_ref[...]
        tx = (-alpha) * v * (p_ref[0] + p_ref[1])
        if has_prev:
            tx = tx - t_ref[...]
        tx_ref[...] = tx
        g_ref[...] = v * tx

    blk = pl.BlockSpec((BN_, LN), lambda i: (i, 0))
    pblk = pl.BlockSpec((2, BN_, LN), lambda i: (0, i, 0))
    in_specs = [pblk, blk] + ([blk] if has_prev else [])
    args = (P, dinv) + ((prev,) if has_prev else ())
    return pl.pallas_call(
        body,
        grid=(NB,),
        in_specs=in_specs,
        out_specs=[blk, blk],
        out_shape=[jax.ShapeDtypeStruct((N, LN), f32)] * 2,
    )(*args)


def _combine4(S, dinv, prev, alpha):
    has_prev = prev is not None

    def body(*refs):
        if has_prev:
            s_ref, v_ref, t_ref, tx_ref, g_ref = refs
        else:
            s_ref, v_ref, tx_ref, g_ref = refs
        v = v_ref[...]
        tx = (-alpha) * v * s_ref[0]
        if has_prev:
            tx = tx - t_ref[0]
        tx_ref[0] = tx
        g_ref[0] = v * tx

    cblk = pl.BlockSpec((1, BN_, LN), lambda c, i: (c, i, 0))
    vblk = pl.BlockSpec((BN_, LN), lambda c, i: (i, 0))
    in_specs = [cblk, vblk] + ([cblk] if has_prev else [])
    args = (S, dinv) + ((prev,) if has_prev else ())
    return pl.pallas_call(
        body,
        grid=(4, NB),
        in_specs=in_specs,
        out_specs=[cblk, cblk],
        out_shape=[jax.ShapeDtypeStruct((4, N, LN), f32)] * 2,
    )(*args)


def _matmul(txs, W, b, c_in, c_out, act_stats):
    nk = len(txs)

    def body(*refs):
        tx_refs = refs[:nk]
        w_ref, b_ref = refs[nk], refs[nk + 1]
        if act_stats:
            z_ref, st_ref, sacc = refs[nk + 2], refs[nk + 3], refs[nk + 4]
        else:
            z_ref = refs[nk + 2]
        z = jnp.broadcast_to(b_ref[...], (BN_, c_out * LN))
        for k in range(nk):
            tx = jnp.concatenate([tx_refs[k][c] for c in range(c_in)], axis=1)
            z = z + jnp.dot(tx, w_ref[k], preferred_element_type=f32)
        if act_stats:
            z = jnp.where(z >= 0.0, z, 0.01 * z)
        for c in range(c_out):
            z_ref[c] = z[:, c * LN:(c + 1) * LN]
        if act_stats:
            i = pl.program_id(0)

            @pl.when(i == 0)
            def _():
                sacc[...] = jnp.zeros_like(sacc)

            sacc[0, :] += jnp.sum(z, axis=0)
            sacc[1, :] += jnp.sum(z * z, axis=0)

            @pl.when(i == NB - 1)
            def _():
                st_ref[...] = sacc[...]

    tblk = pl.BlockSpec((c_in, BN_, LN), lambda i: (0, i, 0))
    wblk = pl.BlockSpec((nk, c_in * LN, c_out * LN), lambda i: (0, 0, 0))
    bblk = pl.BlockSpec((1, c_out * LN), lambda i: (0, 0))
    zblk = pl.BlockSpec((c_out, BN_, LN), lambda i: (0, i, 0))
    out_specs = [zblk]
    out_shape = [jax.ShapeDtypeStruct((c_out, N, LN), f32)]
    scratch = []
    if act_stats:
        out_specs.append(pl.BlockSpec((2, c_out * LN), lambda i: (0, 0)))
        out_shape.append(jax.ShapeDtypeStruct((2, c_out * LN), f32))
        scratch.append(pltpu.VMEM((2, c_out * LN), f32))
    return pl.pallas_call(
        body,
        grid=(NB,),
        in_specs=[tblk] * nk + [wblk, bblk],
        out_specs=out_specs,
        out_shape=out_shape,
        scratch_shapes=scratch,
    )(*txs, W, b)


def _bn_apply(Z, st, gam, bet, dinv):
    def body(z_ref, s_ref, g_ref, b_ref, v_ref, h_ref, o_ref):
        s = s_ref[...]
        mean = s[0] * (1.0 / N)
        var = s[1] * (1.0 / N) - mean * mean
        inv = lax.rsqrt(var + 1e-5) * g_ref[0]
        sh = b_ref[0] - mean * inv
        v = v_ref[...]
        for c in range(4):
            sl = slice(c * LN, (c + 1) * LN)
            h = z_ref[c] * inv[sl] + sh[sl]
            h_ref[c] = h
            o_ref[c] = h * v

    cblk = pl.BlockSpec((4, BN_, LN), lambda i: (0, i, 0))
    sblk = pl.BlockSpec((2, 4 * LN), lambda i: (0, 0))
    pblk = pl.BlockSpec((1, 4 * LN), lambda i: (0, 0))
    vblk = pl.BlockSpec((BN_, LN), lambda i: (i, 0))
    return pl.pallas_call(
        body,
        grid=(NB,),
        in_specs=[cblk, sblk, pblk, pblk, vblk],
        out_specs=[cblk, cblk],
        out_shape=[jax.ShapeDtypeStruct((4, N, LN), f32)] * 2,
    )(Z, st, gam, bet, dinv)


def _final(Z):
    def body(z_ref, o_ref, ssum, smax, smin):
        i = pl.program_id(0)
        z0 = z_ref[0]
        z1 = z_ref[1]
        n2 = (jnp.sum(z0 * z0, axis=1, keepdims=True)
              + jnp.sum(z1 * z1, axis=1, keepdims=True))
        inv = 1.0 / jnp.maximum(jnp.sqrt(n2), 1e-12)
        h0 = z0 * inv
        h1 = z1 * inv

        @pl.when(i == 0)
        def _():
            ssum[...] = jnp.zeros_like(ssum)
            smax[...] = jnp.full_like(smax, -jnp.inf)
            smin[...] = jnp.full_like(smin, jnp.inf)

        ssum[0, :] += jnp.sum(h0, axis=0)
        ssum[1, :] += jnp.sum(h1, axis=0)
        smax[0, :] = jnp.maximum(smax[0, :], jnp.max(h0, axis=0))
        smax[1, :] = jnp.maximum(smax[1, :], jnp.max(h1, axis=0))
        smin[0, :] = jnp.minimum(smin[0, :], jnp.min(h0, axis=0))
        smin[1, :] = jnp.minimum(smin[1, :], jnp.min(h1, axis=0))

        @pl.when(i == NB - 1)
        def _():
            o_ref[0] = ssum[...] * (1.0 / N)
            o_ref[1] = smax[...]
            o_ref[2] = smin[...]
            o_ref[3] = ssum[...]

    return pl.pallas_call(
        body,
        grid=(NB,),
        in_specs=[pl.BlockSpec((2, BN_, LN), lambda i: (0, i, 0))],
        out_specs=pl.BlockSpec((4, 2, LN), lambda i: (0, 0, 0)),
        out_shape=jax.ShapeDtypeStruct((4, 2, LN), f32),
        scratch_shapes=[pltpu.VMEM((2, LN), f32)] * 3,
    )(Z)


def _layer64(Tx0, g0, dinv, srcg, dstm, W, b, c_out, act_stats):
    S0 = _sc_prop4(g0.reshape(4 * N, LN), srcg, dstm).reshape(4, OUTR, LN)[:, :N]
    Tx1, g1 = _combine4(S0, dinv, None, 1.0)
    S1 = _sc_prop4(g1.reshape(4 * N, LN), srcg, dstm).reshape(4, OUTR, LN)[:, :N]
    Tx2, g2 = _combine4(S1, dinv, Tx0, 2.0)
    S2 = _sc_prop4(g2.reshape(4 * N, LN), srcg, dstm).reshape(4, OUTR, LN)[:, :N]
    Tx3, _ = _combine4(S2, dinv, Tx1, 2.0)
    return _matmul([Tx0, Tx1, Tx2, Tx3], W, b, 4, c_out, act_stats)


def kernel(x, edge_index, params):
    p = params
    src = edge_index[0].reshape(E_ROWS, 128)
    dst = edge_index[1].reshape(E_ROWS, 128)
    dstm, srcm = _prep(src, dst)
    padz = jnp.zeros((E_ROWS_PAD - E_ROWS, 128), i32)
    padt = jnp.full((E_ROWS_PAD - E_ROWS, 128), TRASH, i32)
    srcg = jnp.concatenate([src, padz], axis=0)
    dstm = jnp.concatenate([dstm, padt], axis=0)
    srcm = jnp.concatenate([srcm, padt], axis=0)

    degp = _sc_hist(srcm).reshape(2, OUTR, LN)[:, :N]
    dinv = _dinv(degp)

    xpad = jnp.pad(x, ((0, 0), (0, LN - x.shape[1])))
    g0 = _scale(xpad, dinv)
    P0 = _sc_prop1(g0, srcg, dstm).reshape(2, OUTR, LN)[:, :N]
    Tx1, g1 = _combine1(P0, dinv, None, 1.0)
    P1 = _sc_prop1(g1, srcg, dstm).reshape(2, OUTR, LN)[:, :N]
    Tx2, g2 = _combine1(P1, dinv, xpad, 2.0)
    P2 = _sc_prop1(g2, srcg, dstm).reshape(2, OUTR, LN)[:, :N]
    Tx3, _ = _combine1(P2, dinv, Tx1, 2.0)
    W1 = jnp.pad(p["conv1_W"], ((0, 0), (0, LN - x.shape[1]), (0, 0)))
    txs = [a.reshape(1, N, LN) for a in (xpad, Tx1, Tx2, Tx3)]
    Z, st = _matmul(txs, W1, p["conv1_b"].reshape(1, -1), 1, 4, True)
    H, g = _bn_apply(Z, st, p["bn1_g"].reshape(1, -1),
                     p["bn1_b"].reshape(1, -1), dinv)

    Z, st = _layer64(H, g, dinv, srcg, dstm, p["conv2_W"],
                     p["conv2_b"].reshape(1, -1), 4, True)
    H, g = _bn_apply(Z, st, p["bn2_g"].reshape(1, -1),
                     p["bn2_b"].reshape(1, -1), dinv)
    Z, st = _layer64(H, g, dinv, srcg, dstm, p["conv3_W"],
                     p["conv3_b"].reshape(1, -1), 4, True)
    H, g = _bn_apply(Z, st, p["bn3_g"].reshape(1, -1),
                     p["bn3_b"].reshape(1, -1), dinv)

    Z4 = _layer64(H, g, dinv, srcg, dstm, p["conv4_W"],
                  p["conv4_b"].reshape(1, -1), 2, False)[0]
    return _final(Z4).reshape(128)


# speedup vs baseline: 5.2963x; 1.0368x over previous
import functools

import jax
import jax.numpy as jnp
from jax import lax
from jax.experimental import pallas as pl
from jax.experimental.pallas import tpu as pltpu
from jax.experimental.pallas import tpu_sc as plsc

N = 100000
E = 1600000
LN = 16
TRASH = N
ACC_ROWS = 100096
OUTR = N
EG = 128
GRP = 8
E_ROWS = 12500
E_ROWS_PAD = 12800
BN_ = 2000
NB = N // BN_
f32 = jnp.float32
i32 = jnp.int32


def _sc_body(gather, c4, *refs):
    if gather:
        g_hbm, src_hbm, didx_hbm, out_hbm, sidx, didx, rows, acc, gsem, ssem = refs
    else:
        didx_hbm, out_hbm, sidx, didx, rows, acc, gsem, ssem = refs
    cid = lax.axis_index("core")
    sid = lax.axis_index("sub")

    def fill_rows(val):
        def fb(i, _):
            rows[i, :] = val
            return 0
        lax.fori_loop(0, GRP * EG, fb, 0)

    def accumulate(g_tile0, nb, chunk_off):
        def batch(bi, _):
            gb = g_tile0 + bi * GRP
            pltpu.sync_copy(didx_hbm.at[pl.ds(gb, GRP)], didx)
            if gather:
                pltpu.sync_copy(src_hbm.at[pl.ds(gb, GRP)], sidx)
                if c4:
                    for r in range(GRP):
                        for l in range(EG // 16):
                            s = pl.ds(l * 16, 16)
                            sidx[r, s] = sidx[r, s] + chunk_off
                des = [pltpu.async_copy(g_hbm.at[sidx.at[j]],
                                        rows.at[pl.ds(j * EG, EG)], gsem)
                       for j in range(GRP)]
                for d in des:
                    d.wait()
            des = [pltpu.async_copy(rows.at[pl.ds(j * EG, EG)],
                                    acc.at[didx.at[j]], ssem, add=True)
                   for j in range(GRP)]
            for d in des:
                d.wait()
            return 0
        lax.fori_loop(0, nb, batch, 0)

    def do_chunk(g_tile0, nb, chunk_off, out_base):
        fill_rows(jnp.zeros((16,), f32))
        zb = sid * (ACC_ROWS // 16)
        for j in range(6):
            pltpu.sync_copy(rows.at[pl.ds(0, 1024)],
                            acc.at[pl.ds(zb + j * 1024, 1024)])
        pltpu.sync_copy(rows.at[pl.ds(0, ACC_ROWS // 16 - 6 * 1024)],
                        acc.at[pl.ds(zb + 6 * 1024, ACC_ROWS // 16 - 6 * 1024)])
        plsc.subcore_barrier()
        if not gather:
            fill_rows(jnp.ones((16,), f32))
        accumulate(g_tile0, nb, chunk_off)
        plsc.subcore_barrier()
        rb = sid * (OUTR // 16)
        for j in range(6):
            pltpu.sync_copy(acc.at[pl.ds(rb + j * 1024, 1024)],
                            out_hbm.at[pl.ds(out_base + rb + j * 1024, 1024)])
        pltpu.sync_copy(acc.at[pl.ds(rb + 6 * 1024, OUTR // 16 - 6 * 1024)],
                        out_hbm.at[pl.ds(out_base + rb + 6 * 1024,
                                         OUTR // 16 - 6 * 1024)])
        plsc.subcore_barrier()

    if c4:
        g_tile0 = sid * (E_ROWS_PAD // 16)
        for cc in range(2):
            chunk = 2 * cid + cc
            do_chunk(g_tile0, (E_ROWS_PAD // 16) // GRP, chunk * N, chunk * OUTR)
    else:
        g_tile0 = cid * (E_ROWS_PAD // 2) + sid * (E_ROWS_PAD // 32)
        do_chunk(g_tile0, (E_ROWS_PAD // 32) // GRP, 0, cid * OUTR)


def _make_sc(gather, c4):
    out_rows = 4 * OUTR if c4 else 2 * OUTR
    mesh = plsc.VectorSubcoreMesh(core_axis_name="core", subcore_axis_name="sub")
    scratch = [
        pltpu.VMEM((GRP, EG), i32),
        pltpu.VMEM((GRP, EG), i32),
        pltpu.VMEM((GRP * EG, LN), f32),
        pltpu.VMEM_SHARED((ACC_ROWS, LN), f32),
        pltpu.SemaphoreType.DMA,
        pltpu.SemaphoreType.DMA,
    ]
    body = functools.partial(_sc_body, gather, c4)
    return pl.kernel(
        body,
        out_type=jax.ShapeDtypeStruct((out_rows, LN), f32),
        mesh=mesh,
        scratch_types=scratch,
        compiler_params=pltpu.CompilerParams(use_tc_tiling_on_sc=False),
        name=("sc_prop4" if c4 else ("sc_prop1" if gather else "sc_hist")),
    )


_sc_hist = _make_sc(gather=False, c4=False)
_sc_prop1 = _make_sc(gather=True, c4=False)
_sc_prop4 = _make_sc(gather=True, c4=True)


def _prep(src2d, dst2d):
    def body(s_ref, d_ref, dm_ref, sm_ref):
        s = s_ref[...]
        d = d_ref[...]
        m = s != d
        dm_ref[...] = jnp.where(m, d, TRASH)
        sm_ref[...] = jnp.where(m, s, TRASH)
    blk = pl.BlockSpec((E_ROWS, 128), lambda: (0, 0))
    return pl.pallas_call(
        body,
        in_specs=[blk, blk],
        out_specs=[blk, blk],
        out_shape=[jax.ShapeDtypeStruct((E_ROWS, 128), i32)] * 2,
    )(src2d, dst2d)


CB = N // BN_


def _dinv(degp):
    def body(p0_ref, p1_ref, o_ref):
        deg = p0_ref[...] + p1_ref[...]
        o_ref[...] = jnp.where(deg > 0.0, lax.rsqrt(jnp.maximum(deg, 1.0)), 0.0)
    return pl.pallas_call(
        body,
        grid=(NB,),
        in_specs=[pl.BlockSpec((BN_, LN), lambda i: (i, 0)),
                  pl.BlockSpec((BN_, LN), lambda i: (CB + i, 0))],
        out_specs=pl.BlockSpec((BN_, LN), lambda i: (i, 0)),
        out_shape=jax.ShapeDtypeStruct((N, LN), f32),
    )(degp, degp)


def _scale(a, dinv):
    def body(a_ref, v_ref, o_ref):
        o_ref[...] = a_ref[...] * v_ref[...]
    blk = pl.BlockSpec((BN_, LN), lambda i: (i, 0))
    return pl.pallas_call(
        body,
        grid=(NB,),
        in_specs=[blk, blk],
        out_specs=blk,
        out_shape=jax.ShapeDtypeStruct((N, LN), f32),
    )(a, dinv)


def _combine1(P, dinv, prev, alpha):
    has_prev = prev is not None

    def body(*refs):
        if has_prev:
            p0_ref, p1_ref, v_ref, t_ref, tx_ref, g_ref = refs
        else:
            p0_ref, p1_ref, v_ref, tx_ref, g_ref = refs
        v = v_ref[...]
        tx = (-alpha) * v * (p0_ref[...] + p1_ref[...])
        if has_prev:
            tx = tx - t_ref[...]
        tx_ref[...] = tx
        g_ref[...] = v * tx

    blk = pl.BlockSpec((BN_, LN), lambda i: (i, 0))
    p1blk = pl.BlockSpec((BN_, LN), lambda i: (CB + i, 0))
    in_specs = [blk, p1blk, blk] + ([blk] if has_prev else [])
    args = (P, P, dinv) + ((prev,) if has_prev else ())
    return pl.pallas_call(
        body,
        grid=(NB,),
        in_specs=in_specs,
        out_specs=[blk, blk],
        out_shape=[jax.ShapeDtypeStruct((N, LN), f32)] * 2,
    )(*args)


def _combine4(S, dinv, prev, alpha):
    has_prev = prev is not None

    def body(*refs):
        if has_prev:
            s_ref, v_ref, t_ref, tx_ref, g_ref = refs
        else:
            s_ref, v_ref, tx_ref, g_ref = refs
        v = v_ref[...]
        tx = (-alpha) * v * s_ref[...]
        if has_prev:
            tx = tx - t_ref[0]
        tx_ref[0] = tx
        g_ref[...] = v * tx

    sblk = pl.BlockSpec((BN_, LN), lambda c, i: (c * CB + i, 0))
    cblk = pl.BlockSpec((1, BN_, LN), lambda c, i: (c, i, 0))
    vblk = pl.BlockSpec((BN_, LN), lambda c, i: (i, 0))
    in_specs = [sblk, vblk] + ([cblk] if has_prev else [])
    args = (S, dinv) + ((prev,) if has_prev else ())
    return pl.pallas_call(
        body,
        grid=(4, NB),
        in_specs=in_specs,
        out_specs=[cblk, sblk],
        out_shape=[jax.ShapeDtypeStruct((4, N, LN), f32),
                   jax.ShapeDtypeStruct((4 * N, LN), f32)],
    )(*args)


def _matmul(txs, W, b, c_in, c_out, act_stats):
    nk = len(txs)

    def body(*refs):
        tx_refs = refs[:nk]
        w_ref, b_ref = refs[nk], refs[nk + 1]
        if act_stats:
            z_ref, st_ref, sacc = refs[nk + 2], refs[nk + 3], refs[nk + 4]
        else:
            z_ref = refs[nk + 2]
        z = jnp.broadcast_to(b_ref[...], (BN_, c_out * LN))
        for k in range(nk):
            if c_in == 1:
                tx = tx_refs[k][...]
            else:
                tx = jnp.concatenate([tx_refs[k][c] for c in range(c_in)],
                                     axis=1)
            z = z + jnp.dot(tx, w_ref[k], preferred_element_type=f32)
        if act_stats:
            z = jnp.where(z >= 0.0, z, 0.01 * z)
        for c in range(c_out):
            z_ref[c] = z[:, c * LN:(c + 1) * LN]
        if act_stats:
            i = pl.program_id(0)

            @pl.when(i == 0)
            def _():
                sacc[...] = jnp.zeros_like(sacc)

            sacc[0, :] += jnp.sum(z, axis=0)
            sacc[1, :] += jnp.sum(z * z, axis=0)

            @pl.when(i == NB - 1)
            def _():
                st_ref[...] = sacc[...]

    if c_in == 1:
        tblk = pl.BlockSpec((BN_, LN), lambda i: (i, 0))
    else:
        tblk = pl.BlockSpec((c_in, BN_, LN), lambda i: (0, i, 0))
    wblk = pl.BlockSpec((nk, c_in * LN, c_out * LN), lambda i: (0, 0, 0))
    bblk = pl.BlockSpec((1, c_out * LN), lambda i: (0, 0))
    zblk = pl.BlockSpec((c_out, BN_, LN), lambda i: (0, i, 0))
    out_specs = [zblk]
    out_shape = [jax.ShapeDtypeStruct((c_out, N, LN), f32)]
    scratch = []
    if act_stats:
        out_specs.append(pl.BlockSpec((2, c_out * LN), lambda i: (0, 0)))
        out_shape.append(jax.ShapeDtypeStruct((2, c_out * LN), f32))
        scratch.append(pltpu.VMEM((2, c_out * LN), f32))
    return pl.pallas_call(
        body,
        grid=(NB,),
        in_specs=[tblk] * nk + [wblk, bblk],
        out_specs=out_specs,
        out_shape=out_shape,
        scratch_shapes=scratch,
    )(*txs, W, b)


def _bn_apply(Z, st, gam, bet):
    def body(z_ref, s_ref, g_ref, b_ref, h_ref):
        s = s_ref[...]
        mean = s[0] * (1.0 / N)
        var = s[1] * (1.0 / N) - mean * mean
        inv = lax.rsqrt(var + 1e-5) * g_ref[0]
        sh = b_ref[0] - mean * inv
        for c in range(4):
            sl = slice(c * LN, (c + 1) * LN)
            h_ref[c] = z_ref[c] * inv[sl] + sh[sl]

    cblk = pl.BlockSpec((4, BN_, LN), lambda i: (0, i, 0))
    sblk = pl.BlockSpec((2, 4 * LN), lambda i: (0, 0))
    pblk = pl.BlockSpec((1, 4 * LN), lambda i: (0, 0))
    return pl.pallas_call(
        body,
        grid=(NB,),
        in_specs=[cblk, sblk, pblk, pblk],
        out_specs=cblk,
        out_shape=jax.ShapeDtypeStruct((4, N, LN), f32),
    )(Z, st, gam, bet)


def _scale4(H, dinv):
    def body(h_ref, v_ref, o_ref):
        o_ref[...] = h_ref[0] * v_ref[...]
    cblk = pl.BlockSpec((1, BN_, LN), lambda c, i: (c, i, 0))
    vblk = pl.BlockSpec((BN_, LN), lambda c, i: (i, 0))
    oblk = pl.BlockSpec((BN_, LN), lambda c, i: (c * CB + i, 0))
    return pl.pallas_call(
        body,
        grid=(4, NB),
        in_specs=[cblk, vblk],
        out_specs=oblk,
        out_shape=jax.ShapeDtypeStruct((4 * N, LN), f32),
    )(H, dinv)


def _final(Z):
    def body(z_ref, o_ref, ssum, smax, smin):
        i = pl.program_id(0)
        z0 = z_ref[0]
        z1 = z_ref[1]
        n2 = (jnp.sum(z0 * z0, axis=1, keepdims=True)
              + jnp.sum(z1 * z1, axis=1, keepdims=True))
        inv = 1.0 / jnp.maximum(jnp.sqrt(n2), 1e-12)
        h0 = z0 * inv
        h1 = z1 * inv

        @pl.when(i == 0)
        def _():
            ssum[...] = jnp.zeros_like(ssum)
            smax[...] = jnp.full_like(smax, -jnp.inf)
            smin[...] = jnp.full_like(smin, jnp.inf)

        ssum[0, :] += jnp.sum(h0, axis=0)
        ssum[1, :] += jnp.sum(h1, axis=0)
        smax[0, :] = jnp.maximum(smax[0, :], jnp.max(h0, axis=0))
        smax[1, :] = jnp.maximum(smax[1, :], jnp.max(h1, axis=0))
        smin[0, :] = jnp.minimum(smin[0, :], jnp.min(h0, axis=0))
        smin[1, :] = jnp.minimum(smin[1, :], jnp.min(h1, axis=0))

        @pl.when(i == NB - 1)
        def _():
            o_ref[0] = ssum[...] * (1.0 / N)
            o_ref[1] = smax[...]
            o_ref[2] = smin[...]
            o_ref[3] = ssum[...]

    return pl.pallas_call(
        body,
        grid=(NB,),
        in_specs=[pl.BlockSpec((2, BN_, LN), lambda i: (0, i, 0))],
        out_specs=pl.BlockSpec((4, 2, LN), lambda i: (0, 0, 0)),
        out_shape=jax.ShapeDtypeStruct((4, 2, LN), f32),
        scratch_shapes=[pltpu.VMEM((2, LN), f32)] * 3,
    )(Z)


def _layer64(Tx0, g0, dinv, srcg, dstm, W, b, c_out, act_stats):
    S0 = _sc_prop4(g0, srcg, dstm)
    Tx1, g1 = _combine4(S0, dinv, None, 1.0)
    S1 = _sc_prop4(g1, srcg, dstm)
    Tx2, g2 = _combine4(S1, dinv, Tx0, 2.0)
    S2 = _sc_prop4(g2, srcg, dstm)
    Tx3, _ = _combine4(S2, dinv, Tx1, 2.0)
    return _matmul([Tx0, Tx1, Tx2, Tx3], W, b, 4, c_out, act_stats)


def kernel(x, edge_index, params):
    p = params
    src = edge_index[0].reshape(E_ROWS, 128)
    dst = edge_index[1].reshape(E_ROWS, 128)
    dstm, srcm = _prep(src, dst)
    padz = jnp.zeros((E_ROWS_PAD - E_ROWS, 128), i32)
    padt = jnp.full((E_ROWS_PAD - E_ROWS, 128), TRASH, i32)
    srcg = jnp.concatenate([src, padz], axis=0)
    dstm = jnp.concatenate([dstm, padt], axis=0)
    srcm = jnp.concatenate([srcm, padt], axis=0)

    degp = _sc_hist(srcm)
    dinv = _dinv(degp)

    xpad = jnp.pad(x, ((0, 0), (0, LN - x.shape[1])))
    g0 = _scale(xpad, dinv)
    P0 = _sc_prop1(g0, srcg, dstm)
    Tx1, g1 = _combine1(P0, dinv, None, 1.0)
    P1 = _sc_prop1(g1, srcg, dstm)
    Tx2, g2 = _combine1(P1, dinv, xpad, 2.0)
    P2 = _sc_prop1(g2, srcg, dstm)
    Tx3, _ = _combine1(P2, dinv, Tx1, 2.0)
    W1 = jnp.pad(p["conv1_W"], ((0, 0), (0, LN - x.shape[1]), (0, 0)))
    Z, st = _matmul([xpad, Tx1, Tx2, Tx3], W1,
                    p["conv1_b"].reshape(1, -1), 1, 4, True)
    H = _bn_apply(Z, st, p["bn1_g"].reshape(1, -1), p["bn1_b"].reshape(1, -1))
    g = _scale4(H, dinv)

    Z, st = _layer64(H, g, dinv, srcg, dstm, p["conv2_W"],
                     p["conv2_b"].reshape(1, -1), 4, True)
    H = _bn_apply(Z, st, p["bn2_g"].reshape(1, -1), p["bn2_b"].reshape(1, -1))
    g = _scale4(H, dinv)
    Z, st = _layer64(H, g, dinv, srcg, dstm, p["conv3_W"],
                     p["conv3_b"].reshape(1, -1), 4, True)
    H = _bn_apply(Z, st, p["bn3_g"].reshape(1, -1), p["bn3_b"].reshape(1, -1))
    g = _scale4(H, dinv)

    Z4 = _layer64(H, g, dinv, srcg, dstm, p["conv4_W"],
                  p["conv4_b"].reshape(1, -1), 2, False)[0]
    return _final(Z4).reshape(128)


# speedup vs baseline: 5.6247x; 1.0620x over previous
import functools

import jax
import jax.numpy as jnp
from jax import lax
from jax.experimental import pallas as pl
from jax.experimental.pallas import tpu as pltpu
from jax.experimental.pallas import tpu_sc as plsc

N = 100000
E = 1600000
LN = 16
TRASH = N
ACC_ROWS = 100096
OUTR = N
EG = 128
HG = 4
E_ROWS = 12500
E_ROWS_PAD = 12800
BN_ = 2000
NB = N // BN_
f32 = jnp.float32
i32 = jnp.int32


def _sc_body(gather, c4, *refs):
    if gather:
        (g_hbm, src_hbm, didx_hbm, out_hbm, sidxA, didxA, sidxB, didxB,
         rowsA, rowsB, acc, gsemA, gsemB, ssemA, ssemB) = refs
    else:
        (didx_hbm, out_hbm, sidxA, didxA, sidxB, didxB,
         rowsA, rowsB, acc, gsemA, gsemB, ssemA, ssemB) = refs
    cid = lax.axis_index("core")
    sid = lax.axis_index("sub")

    def fill_rows(rows, val):
        def fb(i, _):
            rows[i, :] = val
            return 0
        lax.fori_loop(0, HG * EG, fb, 0)

    def load(sidx, didx, g_tile0, b, chunk_off):
        gb = g_tile0 + b * HG
        pltpu.sync_copy(didx_hbm.at[pl.ds(gb, HG)], didx)
        if gather:
            pltpu.sync_copy(src_hbm.at[pl.ds(gb, HG)], sidx)
            if c4:
                for r in range(HG):
                    for l in range(EG // 16):
                        sl = pl.ds(l * 16, 16)
                        sidx[r, sl] = sidx[r, sl] + chunk_off

    def fire_g(sidx, rows, gsem):
        for j in range(HG):
            pltpu.async_copy(g_hbm.at[sidx.at[j]],
                             rows.at[pl.ds(j * EG, EG)], gsem)

    def drain_g(sidx, rows, gsem):
        for j in range(HG):
            pltpu.make_async_copy(g_hbm.at[sidx.at[j]],
                                  rows.at[pl.ds(j * EG, EG)], gsem).wait()

    def fire_s(didx, rows, ssem):
        for j in range(HG):
            pltpu.async_copy(rows.at[pl.ds(j * EG, EG)],
                             acc.at[didx.at[j]], ssem, add=True)

    def drain_s(didx, rows, ssem):
        for j in range(HG):
            pltpu.make_async_copy(rows.at[pl.ds(j * EG, EG)],
                                  acc.at[didx.at[j]], ssem).wait()

    def accumulate(g_tile0, nbatch, chunk_off):
        if gather:
            load(sidxA, didxA, g_tile0, 0, chunk_off)
            fire_g(sidxA, rowsA, gsemA)

            def pair(i, _):
                b = 2 * i
                load(sidxB, didxB, g_tile0, b + 1, chunk_off)
                fire_g(sidxB, rowsB, gsemB)
                drain_g(sidxA, rowsA, gsemA)
                fire_s(didxA, rowsA, ssemA)
                drain_g(sidxB, rowsB, gsemB)
                fire_s(didxB, rowsB, ssemB)
                drain_s(didxA, rowsA, ssemA)
                load(sidxA, didxA, g_tile0, b + 2, chunk_off)
                fire_g(sidxA, rowsA, gsemA)
                drain_s(didxB, rowsB, ssemB)
                return 0
            lax.fori_loop(0, nbatch // 2 - 1, pair, 0)
            b = nbatch - 2
            load(sidxB, didxB, g_tile0, b + 1, chunk_off)
            fire_g(sidxB, rowsB, gsemB)
            drain_g(sidxA, rowsA, gsemA)
            fire_s(didxA, rowsA, ssemA)
            drain_g(sidxB, rowsB, gsemB)
            fire_s(didxB, rowsB, ssemB)
            drain_s(didxA, rowsA, ssemA)
            drain_s(didxB, rowsB, ssemB)
        else:
            def batch(bi, _):
                load(sidxA, didxA, g_tile0, bi, chunk_off)
                fire_s(didxA, rowsA, ssemA)
                drain_s(didxA, rowsA, ssemA)
                return 0
            lax.fori_loop(0, nbatch, batch, 0)

    def do_chunk(g_tile0, nbatch, chunk_off, out_base):
        fill_rows(rowsA, jnp.zeros((16,), f32))
        zb = sid * (ACC_ROWS // 16)
        for j in range(12):
            pltpu.sync_copy(rowsA.at[pl.ds(0, 512)],
                            acc.at[pl.ds(zb + j * 512, 512)])
        pltpu.sync_copy(rowsA.at[pl.ds(0, ACC_ROWS // 16 - 12 * 512)],
                        acc.at[pl.ds(zb + 12 * 512, ACC_ROWS // 16 - 12 * 512)])
        plsc.subcore_barrier()
        if not gather:
            fill_rows(rowsA, jnp.ones((16,), f32))
        accumulate(g_tile0, nbatch, chunk_off)
        plsc.subcore_barrier()
        rb = sid * (OUTR // 16)
        for j in range(6):
            pltpu.sync_copy(acc.at[pl.ds(rb + j * 1024, 1024)],
                            out_hbm.at[pl.ds(out_base + rb + j * 1024, 1024)])
        pltpu.sync_copy(acc.at[pl.ds(rb + 6 * 1024, OUTR // 16 - 6 * 1024)],
                        out_hbm.at[pl.ds(out_base + rb + 6 * 1024,
                                         OUTR // 16 - 6 * 1024)])
        plsc.subcore_barrier()

    if c4:
        g_tile0 = sid * (E_ROWS_PAD // 16)
        for cc in range(2):
            chunk = 2 * cid + cc
            do_chunk(g_tile0, (E_ROWS_PAD // 16) // HG, chunk * N, chunk * OUTR)
    else:
        g_tile0 = cid * (E_ROWS_PAD // 2) + sid * (E_ROWS_PAD // 32)
        do_chunk(g_tile0, (E_ROWS_PAD // 32) // HG, 0, cid * OUTR)


def _make_sc(gather, c4):
    out_rows = 4 * OUTR if c4 else 2 * OUTR
    mesh = plsc.VectorSubcoreMesh(core_axis_name="core", subcore_axis_name="sub")
    scratch = [
        pltpu.VMEM((HG, EG), i32),
        pltpu.VMEM((HG, EG), i32),
        pltpu.VMEM((HG, EG), i32),
        pltpu.VMEM((HG, EG), i32),
        pltpu.VMEM((HG * EG, LN), f32),
        pltpu.VMEM((HG * EG, LN), f32),
        pltpu.VMEM_SHARED((ACC_ROWS, LN), f32),
        pltpu.SemaphoreType.DMA,
        pltpu.SemaphoreType.DMA,
        pltpu.SemaphoreType.DMA,
        pltpu.SemaphoreType.DMA,
    ]
    body = functools.partial(_sc_body, gather, c4)
    return pl.kernel(
        body,
        out_type=jax.ShapeDtypeStruct((out_rows, LN), f32),
        mesh=mesh,
        scratch_types=scratch,
        compiler_params=pltpu.CompilerParams(use_tc_tiling_on_sc=False),
        name=("sc_prop4" if c4 else ("sc_prop1" if gather else "sc_hist")),
    )


_sc_hist = _make_sc(gather=False, c4=False)
_sc_prop1 = _make_sc(gather=True, c4=False)
_sc_prop4 = _make_sc(gather=True, c4=True)


def _prep(src2d, dst2d):
    def body(s_ref, d_ref, dm_ref, sm_ref):
        s = s_ref[...]
        d = d_ref[...]
        m = s != d
        dm_ref[...] = jnp.where(m, d, TRASH)
        sm_ref[...] = jnp.where(m, s, TRASH)
    blk = pl.BlockSpec((E_ROWS, 128), lambda: (0, 0))
    return pl.pallas_call(
        body,
        in_specs=[blk, blk],
        out_specs=[blk, blk],
        out_shape=[jax.ShapeDtypeStruct((E_ROWS, 128), i32)] * 2,
    )(src2d, dst2d)


CB = N // BN_


def _dinv(degp):
    def body(p0_ref, p1_ref, o_ref):
        deg = p0_ref[...] + p1_ref[...]
        o_ref[...] = jnp.where(deg > 0.0, lax.rsqrt(jnp.maximum(deg, 1.0)), 0.0)
    return pl.pallas_call(
        body,
        grid=(NB,),
        in_specs=[pl.BlockSpec((BN_, LN), lambda i: (i, 0)),
                  pl.BlockSpec((BN_, LN), lambda i: (CB + i, 0))],
        out_specs=pl.BlockSpec((BN_, LN), lambda i: (i, 0)),
        out_shape=jax.ShapeDtypeStruct((N, LN), f32),
    )(degp, degp)


def _scale(a, dinv):
    def body(a_ref, v_ref, o_ref):
        o_ref[...] = a_ref[...] * v_ref[...]
    blk = pl.BlockSpec((BN_, LN), lambda i: (i, 0))
    return pl.pallas_call(
        body,
        grid=(NB,),
        in_specs=[blk, blk],
        out_specs=blk,
        out_shape=jax.ShapeDtypeStruct((N, LN), f32),
    )(a, dinv)


def _combine1(P, dinv, prev, alpha):
    has_prev = prev is not None

    def body(*refs):
        if has_prev:
            p0_ref, p1_ref, v_ref, t_ref, tx_ref, g_ref = refs
        else:
            p0_ref, p1_ref, v_ref, tx_ref, g_ref = refs
        v = v_ref[...]
        tx = (-alpha) * v * (p0_ref[...] + p1_ref[...])
        if has_prev:
            tx = tx - t_ref[...]
        tx_ref[...] = tx
        g_ref[...] = v * tx

    blk = pl.BlockSpec((BN_, LN), lambda i: (i, 0))
    p1blk = pl.BlockSpec((BN_, LN), lambda i: (CB + i, 0))
    in_specs = [blk, p1blk, blk] + ([blk] if has_prev else [])
    args = (P, P, dinv) + ((prev,) if has_prev else ())
    return pl.pallas_call(
        body,
        grid=(NB,),
        in_specs=in_specs,
        out_specs=[blk, blk],
        out_shape=[jax.ShapeDtypeStruct((N, LN), f32)] * 2,
    )(*args)


def _combine4(S, dinv, prev, alpha):
    has_prev = prev is not None

    def body(*refs):
        if has_prev:
            s_ref, v_ref, t_ref, tx_ref, g_ref = refs
        else:
            s_ref, v_ref, tx_ref, g_ref = refs
        v = v_ref[...]
        tx = (-alpha) * v * s_ref[...]
        if has_prev:
            tx = tx - t_ref[0]
        tx_ref[0] = tx
        g_ref[...] = v * tx

    sblk = pl.BlockSpec((BN_, LN), lambda c, i: (c * CB + i, 0))
    cblk = pl.BlockSpec((1, BN_, LN), lambda c, i: (c, i, 0))
    vblk = pl.BlockSpec((BN_, LN), lambda c, i: (i, 0))
    in_specs = [sblk, vblk] + ([cblk] if has_prev else [])
    args = (S, dinv) + ((prev,) if has_prev else ())
    return pl.pallas_call(
        body,
        grid=(4, NB),
        in_specs=in_specs,
        out_specs=[cblk, sblk],
        out_shape=[jax.ShapeDtypeStruct((4, N, LN), f32),
                   jax.ShapeDtypeStruct((4 * N, LN), f32)],
    )(*args)


def _matmul(txs, W, b, c_in, c_out, act_stats):
    nk = len(txs)

    def body(*refs):
        tx_refs = refs[:nk]
        w_ref, b_ref = refs[nk], refs[nk + 1]
        if act_stats:
            z_ref, st_ref, sacc = refs[nk + 2], refs[nk + 3], refs[nk + 4]
        else:
            z_ref = refs[nk + 2]
        z = jnp.broadcast_to(b_ref[...], (BN_, c_out * LN))
        for k in range(nk):
            if c_in == 1:
                tx = tx_refs[k][...]
            else:
                tx = jnp.concatenate([tx_refs[k][c] for c in range(c_in)],
                                     axis=1)
            z = z + jnp.dot(tx, w_ref[k], preferred_element_type=f32)
        if act_stats:
            z = jnp.where(z >= 0.0, z, 0.01 * z)
        for c in range(c_out):
            z_ref[c] = z[:, c * LN:(c + 1) * LN]
        if act_stats:
            i = pl.program_id(0)

            @pl.when(i == 0)
            def _():
                sacc[...] = jnp.zeros_like(sacc)

            sacc[0, :] += jnp.sum(z, axis=0)
            sacc[1, :] += jnp.sum(z * z, axis=0)

            @pl.when(i == NB - 1)
            def _():
                st_ref[...] = sacc[...]

    if c_in == 1:
        tblk = pl.BlockSpec((BN_, LN), lambda i: (i, 0))
    else:
        tblk = pl.BlockSpec((c_in, BN_, LN), lambda i: (0, i, 0))
    wblk = pl.BlockSpec((nk, c_in * LN, c_out * LN), lambda i: (0, 0, 0))
    bblk = pl.BlockSpec((1, c_out * LN), lambda i: (0, 0))
    zblk = pl.BlockSpec((c_out, BN_, LN), lambda i: (0, i, 0))
    out_specs = [zblk]
    out_shape = [jax.ShapeDtypeStruct((c_out, N, LN), f32)]
    scratch = []
    if act_stats:
        out_specs.append(pl.BlockSpec((2, c_out * LN), lambda i: (0, 0)))
        out_shape.append(jax.ShapeDtypeStruct((2, c_out * LN), f32))
        scratch.append(pltpu.VMEM((2, c_out * LN), f32))
    return pl.pallas_call(
        body,
        grid=(NB,),
        in_specs=[tblk] * nk + [wblk, bblk],
        out_specs=out_specs,
        out_shape=out_shape,
        scratch_shapes=scratch,
    )(*txs, W, b)


def _bn_apply(Z, st, gam, bet):
    def body(z_ref, s_ref, g_ref, b_ref, h_ref):
        s = s_ref[...]
        mean = s[0] * (1.0 / N)
        var = s[1] * (1.0 / N) - mean * mean
        inv = lax.rsqrt(var + 1e-5) * g_ref[0]
        sh = b_ref[0] - mean * inv
        for c in range(4):
            sl = slice(c * LN, (c + 1) * LN)
            h_ref[c] = z_ref[c] * inv[sl] + sh[sl]

    cblk = pl.BlockSpec((4, BN_, LN), lambda i: (0, i, 0))
    sblk = pl.BlockSpec((2, 4 * LN), lambda i: (0, 0))
    pblk = pl.BlockSpec((1, 4 * LN), lambda i: (0, 0))
    return pl.pallas_call(
        body,
        grid=(NB,),
        in_specs=[cblk, sblk, pblk, pblk],
        out_specs=cblk,
        out_shape=jax.ShapeDtypeStruct((4, N, LN), f32),
    )(Z, st, gam, bet)


def _scale4(H, dinv):
    def body(h_ref, v_ref, o_ref):
        o_ref[...] = h_ref[0] * v_ref[...]
    cblk = pl.BlockSpec((1, BN_, LN), lambda c, i: (c, i, 0))
    vblk = pl.BlockSpec((BN_, LN), lambda c, i: (i, 0))
    oblk = pl.BlockSpec((BN_, LN), lambda c, i: (c * CB + i, 0))
    return pl.pallas_call(
        body,
        grid=(4, NB),
        in_specs=[cblk, vblk],
        out_specs=oblk,
        out_shape=jax.ShapeDtypeStruct((4 * N, LN), f32),
    )(H, dinv)


def _final(Z):
    def body(z_ref, o_ref, ssum, smax, smin):
        i = pl.program_id(0)
        z0 = z_ref[0]
        z1 = z_ref[1]
        n2 = (jnp.sum(z0 * z0, axis=1, keepdims=True)
              + jnp.sum(z1 * z1, axis=1, keepdims=True))
        inv = 1.0 / jnp.maximum(jnp.sqrt(n2), 1e-12)
        h0 = z0 * inv
        h1 = z1 * inv

        @pl.when(i == 0)
        def _():
            ssum[...] = jnp.zeros_like(ssum)
            smax[...] = jnp.full_like(smax, -jnp.inf)
            smin[...] = jnp.full_like(smin, jnp.inf)

        ssum[0, :] += jnp.sum(h0, axis=0)
        ssum[1, :] += jnp.sum(h1, axis=0)
        smax[0, :] = jnp.maximum(smax[0, :], jnp.max(h0, axis=0))
        smax[1, :] = jnp.maximum(smax[1, :], jnp.max(h1, axis=0))
        smin[0, :] = jnp.minimum(smin[0, :], jnp.min(h0, axis=0))
        smin[1, :] = jnp.minimum(smin[1, :], jnp.min(h1, axis=0))

        @pl.when(i == NB - 1)
        def _():
            o_ref[0] = ssum[...] * (1.0 / N)
            o_ref[1] = smax[...]
            o_ref[2] = smin[...]
            o_ref[3] = ssum[...]

    return pl.pallas_call(
        body,
        grid=(NB,),
        in_specs=[pl.BlockSpec((2, BN_, LN), lambda i: (0, i, 0))],
        out_specs=pl.BlockSpec((4, 2, LN), lambda i: (0, 0, 0)),
        out_shape=jax.ShapeDtypeStruct((4, 2, LN), f32),
        scratch_shapes=[pltpu.VMEM((2, LN), f32)] * 3,
    )(Z)


def _layer64(Tx0, g0, dinv, srcg, dstm, W, b, c_out, act_stats):
    S0 = _sc_prop4(g0, srcg, dstm)
    Tx1, g1 = _combine4(S0, dinv, None, 1.0)
    S1 = _sc_prop4(g1, srcg, dstm)
    Tx2, g2 = _combine4(S1, dinv, Tx0, 2.0)
    S2 = _sc_prop4(g2, srcg, dstm)
    Tx3, _ = _combine4(S2, dinv, Tx1, 2.0)
    return _matmul([Tx0, Tx1, Tx2, Tx3], W, b, 4, c_out, act_stats)


def kernel(x, edge_index, params):
    p = params
    src = edge_index[0].reshape(E_ROWS, 128)
    dst = edge_index[1].reshape(E_ROWS, 128)
    dstm, srcm = _prep(src, dst)
    padz = jnp.zeros((E_ROWS_PAD - E_ROWS, 128), i32)
    padt = jnp.full((E_ROWS_PAD - E_ROWS, 128), TRASH, i32)
    srcg = jnp.concatenate([src, padz], axis=0)
    dstm = jnp.concatenate([dstm, padt], axis=0)
    srcm = jnp.concatenate([srcm, padt], axis=0)

    degp = _sc_hist(srcm)
    dinv = _dinv(degp)

    xpad = jnp.pad(x, ((0, 0), (0, LN - x.shape[1])))
    g0 = _scale(xpad, dinv)
    P0 = _sc_prop1(g0, srcg, dstm)
    Tx1, g1 = _combine1(P0, dinv, None, 1.0)
    P1 = _sc_prop1(g1, srcg, dstm)
    Tx2, g2 = _combine1(P1, dinv, xpad, 2.0)
    P2 = _sc_prop1(g2, srcg, dstm)
    Tx3, _ = _combine1(P2, dinv, Tx1, 2.0)
    W1 = jnp.pad(p["conv1_W"], ((0, 0), (0, LN - x.shape[1]), (0, 0)))
    Z, st = _matmul([xpad, Tx1, Tx2, Tx3], W1,
                    p["conv1_b"].reshape(1, -1), 1, 4, True)
    H = _bn_apply(Z, st, p["bn1_g"].reshape(1, -1), p["bn1_b"].reshape(1, -1))
    g = _scale4(H, dinv)

    Z, st = _layer64(H, g, dinv, srcg, dstm, p["conv2_W"],
                     p["conv2_b"].reshape(1, -1), 4, True)
    H = _bn_apply(Z, st, p["bn2_g"].reshape(1, -1), p["bn2_b"].reshape(1, -1))
    g = _scale4(H, dinv)
    Z, st = _layer64(H, g, dinv, srcg, dstm, p["conv3_W"],
                     p["conv3_b"].reshape(1, -1), 4, True)
    H = _bn_apply(Z, st, p["bn3_g"].reshape(1, -1), p["bn3_b"].reshape(1, -1))
    g = _scale4(H, dinv)

    Z4 = _layer64(H, g, dinv, srcg, dstm, p["conv4_W"],
                  p["conv4_b"].reshape(1, -1), 2, False)[0]
    return _final(Z4).reshape(128)
